# baseline (speedup 1.0000x reference)
import functools

import jax
import jax.numpy as jnp
from jax import lax
from jax.experimental import pallas as pl
from jax.experimental.pallas import tpu as pltpu
from jax.experimental.pallas import tpu_sc as plsc


def _plan(n, nw):
    best = None
    for c in range(1, 64):
        t = -(-n // (nw * c))
        t = -(-t // 8) * 8
        if t > 128:
            continue
        key = (c, nw * c * t)
        if best is None or key < best[0]:
            best = (key, (c, t))
    return best[1]


@functools.lru_cache(maxsize=None)
def _build(grad_sizes, idx_sizes):
    info = plsc.get_sparse_core_info()
    nw = info.num_cores * info.num_subcores
    nc = info.num_cores
    plans = [_plan(n, nw) for n in idx_sizes]
    offs = []
    o = 0
    for c, t in plans:
        offs.append(o)
        o += c * t
    width = o

    def body(g0, g1, g2, g3, ih, oh, iv, vv, s0, s1, s2, s3, sw):
        gs = (g0, g1, g2, g3)
        sems = (s0, s1, s2, s3)
        w = lax.axis_index("s") * nc + lax.axis_index("c")
        wbase = pl.multiple_of(w * width, 8)
        pltpu.sync_copy(ih.at[pl.ds(wbase, width)], iv)
        descs = []
        for g, off, (c, t) in zip(gs, offs, plans):
            for j in range(c):
                sl = pl.ds(off + j * t, t)
                descs.append(pltpu.async_copy(g.at[iv.at[sl]], vv.at[sl], s0))
        for d in descs:
            d.wait()
        pltpu.sync_copy(vv, oh.at[pl.ds(wbase, width)])

    kfn = pl.kernel(
        body,
        out_type=jax.ShapeDtypeStruct((nw * width,), jnp.float32),
        mesh=plsc.VectorSubcoreMesh(core_axis_name="c", subcore_axis_name="s"),
        scratch_types=[
            pltpu.VMEM((width,), jnp.int32),
            pltpu.VMEM((width,), jnp.float32),
            pltpu.SemaphoreType.DMA,
            pltpu.SemaphoreType.DMA,
            pltpu.SemaphoreType.DMA,
            pltpu.SemaphoreType.DMA,
            pltpu.SemaphoreType.DMA,
        ],
    )
    return kfn, plans, offs, nw


def _tile_view(g):
    if g.ndim == 1:
        return g, None
    r, c = g.shape
    if r % 8 == 0 and c % 128 == 0:
        v = g.reshape(r // 8, 8, c // 128, 128).transpose(0, 2, 1, 3)
        return v.reshape(-1), c
    return g.reshape(-1), None


def _phys_idx(idx, c):
    if c is None:
        return idx
    r_i = idx // c
    c_i = idx - r_i * c
    tile = (r_i >> 3) * (c >> 7) + (c_i >> 7)
    return (tile << 10) + ((r_i & 7) << 7) + (c_i & 127)


def kernel(grad_0, grad_1, grad_2, grad_3,
           indices_0, indices_1, indices_2, indices_3):
    views = [_tile_view(g) for g in (grad_0, grad_1, grad_2, grad_3)]
    grads = [v for v, _ in views]
    idxs = [indices_0, indices_1, indices_2, indices_3]
    ns = tuple(int(i.shape[0]) for i in idxs)
    kfn, plans, offs, nw = _build(tuple(int(g.shape[0]) for g in grads), ns)
    cols = []
    for idx, (_, cdim), (c, t) in zip(idxs, views, plans):
        p = nw * c * t
        i32 = _phys_idx(idx.astype(jnp.int32), cdim)
        cols.append(jnp.pad(i32, (0, p - i32.shape[0])).reshape(nw, c * t))
    idx_cat = jnp.concatenate(cols, axis=1).reshape(-1)
    width = idx_cat.shape[0] // nw
    out = kfn(*grads, idx_cat).reshape(nw, width)
    parts = [
        lax.slice(out, (0, off), (nw, off + c * t)).reshape(-1)[:n]
        for off, (c, t), n in zip(offs, plans, ns)
    ]
    return jnp.concatenate(parts).reshape(1, -1)


# speedup vs baseline: 1.1921x; 1.1921x over previous
import functools

import jax
import jax.numpy as jnp
from jax import lax
from jax.experimental import pallas as pl
from jax.experimental.pallas import tpu as pltpu
from jax.experimental.pallas import tpu_sc as plsc


def _plan(n, nw):
    best = None
    for c in range(1, 64):
        t = -(-n // (nw * c))
        t = -(-t // 8) * 8
        if t > 128:
            continue
        key = (nw * c * t, c)
        if best is None or key < best[0]:
            best = (key, (c, t))
    return best[1]


@functools.lru_cache(maxsize=None)
def _build(grad_sizes, idx_sizes):
    info = plsc.get_sparse_core_info()
    nw = info.num_cores * info.num_subcores
    nc = info.num_cores
    plans = [_plan(n, nw) for n in idx_sizes]
    offs = []
    o = 0
    for c, t in plans:
        offs.append(o)
        o += c * t
    width = o

    def body(g0, g1, g2, g3, ih, oh, iv, vv, s0):
        gs = (g0, g1, g2, g3)
        w = lax.axis_index("s") * nc + lax.axis_index("c")
        pltpu.sync_copy(ih.at[w], iv)
        descs = []
        for g, off, (c, t) in zip(gs, offs, plans):
            for j in range(c):
                sl = pl.ds(off + j * t, t)
                descs.append(pltpu.async_copy(g.at[iv.at[sl]], vv.at[sl], s0))
        for d in descs:
            d.wait()
        pltpu.sync_copy(vv, oh.at[w])

    kfn = pl.kernel(
        body,
        out_type=jax.ShapeDtypeStruct((nw, width), jnp.float32),
        mesh=plsc.VectorSubcoreMesh(core_axis_name="c", subcore_axis_name="s"),
        scratch_types=[
            pltpu.VMEM((width,), jnp.int32),
            pltpu.VMEM((width,), jnp.float32),
            pltpu.SemaphoreType.DMA,
        ],
    )
    return kfn, plans, offs, nw


def _tile_view(g):
    if g.ndim == 1:
        return g, None
    r, c = g.shape
    if r % 8 == 0 and c % 128 == 0:
        v = g.reshape(r // 8, 8, c // 128, 128).transpose(0, 2, 1, 3)
        return v.reshape(-1), c
    return g.reshape(-1), None


def _phys_idx(idx, c):
    if c is None:
        return idx
    r_i = idx // c
    c_i = idx - r_i * c
    tile = (r_i >> 3) * (c >> 7) + (c_i >> 7)
    return (tile << 10) + ((r_i & 7) << 7) + (c_i & 127)


def kernel(grad_0, grad_1, grad_2, grad_3,
           indices_0, indices_1, indices_2, indices_3):
    views = [_tile_view(g) for g in (grad_0, grad_1, grad_2, grad_3)]
    grads = [v for v, _ in views]
    idxs = [indices_0, indices_1, indices_2, indices_3]
    ns = tuple(int(i.shape[0]) for i in idxs)
    kfn, plans, offs, nw = _build(tuple(int(g.shape[0]) for g in grads), ns)
    cols = []
    for idx, (_, cdim), (c, t) in zip(idxs, views, plans):
        p = nw * c * t
        i32 = _phys_idx(idx.astype(jnp.int32), cdim)
        cols.append(jnp.pad(i32, (0, p - i32.shape[0])).reshape(nw, c * t))
    idx_cat = jnp.concatenate(cols, axis=1)
    out = kfn(*grads, idx_cat)
    parts = [
        lax.slice(out, (0, off), (nw, off + c * t)).reshape(-1)[:n]
        for off, (c, t), n in zip(offs, plans, ns)
    ]
    return jnp.concatenate(parts).reshape(1, -1)
